# baseline (speedup 1.0000x reference)
import functools

import jax
import jax.numpy as jnp
from jax import lax
from jax.experimental import pallas as pl
from jax.experimental.pallas import tpu as pltpu
from jax.experimental.pallas import tpu_sc as plsc

NC = 2
NS = 16
NW = NC * NS
LANES = 16
NBUF = 5
AHEAD = NBUF - 2
TILE_D = 8
TILE_B = 128
PAD_B = TILE_B + 1


def _build_emb_kernel(B, L, D, V):
    BPW = B // NW
    DG = D // TILE_D
    NTILE = DG * (B // TILE_B)
    NBG = TILE_B // LANES

    mesh = plsc.VectorSubcoreMesh(core_axis_name="c", subcore_axis_name="s")

    @functools.partial(
        pl.kernel,
        out_type=jax.ShapeDtypeStruct((L, NTILE, TILE_D, TILE_B), jnp.float32),
        mesh=mesh,
        compiler_params=pltpu.CompilerParams(
            use_tc_tiling_on_sc=False, needs_layout_passes=False),
        scratch_types=[
            pltpu.VMEM((L + AHEAD, TILE_B), jnp.int32),
            pltpu.VMEM((BPW,), jnp.int32),
            pltpu.VMEM((LANES + TILE_B,), jnp.float32),
            pltpu.VMEM((NBUF * TILE_B, D), jnp.float32),
            pltpu.VMEM((NBUF, DG, TILE_D, PAD_B), jnp.float32),
        ] + [pltpu.SemaphoreType.DMA] * (2 * NBUF),
    )
    def body(sent_ref, len_ref, table_ref, out_ref, idx_v, lens_v, mask_v,
             rows_v, ow_v, *sems):
        gsem = sems[:NBUF]
        osem = sems[NBUF:]
        wid = lax.axis_index("s") * NC + lax.axis_index("c")
        pltpu.sync_copy(sent_ref.at[:, pl.ds(wid * BPW, BPW)],
                        idx_v.at[pl.ds(0, L)])
        pltpu.sync_copy(len_ref.at[pl.ds(wid * BPW, BPW)], lens_v)
        zi = jnp.full((LANES,), 0, jnp.int32)
        for k in range(AHEAD):
            for g in range(TILE_B // LANES):
                idx_v[L + k, pl.ds(g * LANES, LANES)] = zi

        def start_gather(l, b):
            pltpu.async_copy(table_ref.at[idx_v.at[l]],
                             rows_v.at[pl.ds(b * TILE_B, TILE_B)], gsem[b])

        def wait_gather(b):
            pltpu.make_async_copy(table_ref.at[idx_v.at[0]],
                                  rows_v.at[pl.ds(b * TILE_B, TILE_B)],
                                  gsem[b]).wait()

        def start_out(l, b):
            for dg in range(DG):
                pltpu.async_copy(ow_v.at[b, dg, :, pl.ds(0, TILE_B)],
                                 out_ref.at[l, dg * (B // TILE_B) + wid],
                                 osem[b])

        def wait_out(b):
            for dg in range(DG):
                pltpu.make_async_copy(ow_v.at[b, dg, :, pl.ds(0, TILE_B)],
                                      out_ref.at[0, dg * (B // TILE_B) + wid],
                                      osem[b]).wait()

        iota = lax.iota(jnp.int32, LANES)
        fs_vec = lax.bitwise_and(iota, jnp.full((LANES,), TILE_D - 1,
                                                jnp.int32))
        dghalf = lax.shift_right_logical(
            iota, jnp.full((LANES,), 3, jnp.int32))

        def mask_transpose(l, b):
            lv = jnp.full((LANES,), l, jnp.int32)
            for bg in range(NBG):
                lens16 = lens_v[pl.ds(bg * LANES, LANES)]
                m = (lv < lens16).astype(jnp.float32)
                mask_v[pl.ds(LANES + bg * LANES, LANES)] = m
            bsplat = jnp.full((LANES,), b, jnp.int32)
            for r in range(TILE_B):
                mv = plsc.load_gather(
                    mask_v, [jnp.full((LANES,), LANES + r, jnp.int32)])
                rsplat = jnp.full((LANES,), r, jnp.int32)
                for h in range(D // LANES):
                    v = rows_v[b * TILE_B + r, pl.ds(h * LANES, LANES)] * mv
                    dgv = dghalf + jnp.full((LANES,), h * (LANES // TILE_D),
                                            jnp.int32)
                    plsc.store_scatter(ow_v, [bsplat, dgv, fs_vec, rsplat], v)

        for k in range(AHEAD):
            start_gather(k, k)

        def quad_body(l4, carry):
            for bi in range(NBUF):
                l = l4 * NBUF + bi
                wait_gather(bi)
                mask_transpose(l, bi)
                nb = (bi + AHEAD) % NBUF
                if bi >= 2:
                    wait_out(nb)
                else:
                    @pl.when(l4 > 0)
                    def _():
                        wait_out(nb)

                start_gather(l + AHEAD, nb)
                start_out(l, bi)
            return carry

        lax.fori_loop(0, L // NBUF, quad_body, 0)
        for k in range(AHEAD):
            wait_gather((L + k) % NBUF)
        wait_out((L - 2) % NBUF)
        wait_out((L - 1) % NBUF)

    return body


def kernel(sentences, sent_lengths, table):
    B, L = sentences.shape
    V, D = table.shape
    out_pl = _build_emb_kernel(B, L, D, V)(
        sentences.T, sent_lengths, table)
    out = out_pl.reshape(L, D // TILE_D, B // TILE_B, TILE_D, TILE_B)
    out = out.transpose(2, 4, 0, 1, 3).reshape(B, L, D)
    return out
